# Initial kernel scaffold; baseline (speedup 1.0000x reference)
import jax
import jax.numpy as jnp
from jax.experimental import pallas as pl


def kernel(x, W_embed, b_embed, W_cls, b_cls):
    raise NotImplementedError("write your pallas kernel here")


# speedup vs baseline: 1.5361x; 1.5361x over previous
import jax
import jax.numpy as jnp
from jax.experimental import pallas as pl
from jax.experimental.pallas import tpu as pltpu

B = 8
T = 2048
F = 2048
C = 20
K_EASY = T // 5
K_HARD = T // 20
CH = 512
NCH = T // CH
IMIN = -2**31


def _imin():
    return jnp.full((), IMIN, jnp.int32)


def _tree_reduce_c(cas):
    lvl = [cas[:, :, i] for i in range(C)]
    while len(lvl) > 1:
        if len(lvl) % 2:
            lvl.append(jnp.zeros_like(lvl[0]))
        lvl = [lvl[2 * i] + lvl[2 * i + 1] for i in range(len(lvl) // 2)]
    return lvl[0]


def _shift0(v, s):
    if s > 0:
        return jnp.concatenate([jnp.zeros((v.shape[0], s), v.dtype), v[:, :-s]], axis=1)
    if s < 0:
        return jnp.concatenate([v[:, -s:], jnp.zeros((v.shape[0], -s), v.dtype)], axis=1)
    return v


def _erode(b, L):
    p = L // 2
    m = b
    for d in range(-p, p + 1):
        if d != 0:
            m = jnp.minimum(m, _shift0(b, d))
    return m


def _dilate(b, L):
    p = L // 2
    m = b
    for d in range(-p, p + 1):
        if d != 0:
            m = jnp.maximum(m, _shift0(b, d))
    return m


def _rank_chunk(key_row, i0, need_asc):
    ai = jnp.transpose(key_row[:, i0:i0 + CH])
    jio = jax.lax.broadcasted_iota(jnp.int32, (CH, T), 1)
    iio = jax.lax.broadcasted_iota(jnp.int32, (CH, T), 0) + i0
    eqb = jnp.where((key_row == ai) & (jio < iio), 1.0, 0.0)
    gt = jnp.where(key_row > ai, 1.0, 0.0)
    rd = jnp.sum(gt + eqb, axis=1).reshape(1, CH)
    if not need_asc:
        return rd, None
    lt = jnp.where(key_row < ai, 1.0, 0.0)
    ra = jnp.sum(lt + eqb, axis=1).reshape(1, CH)
    return rd, ra


def _topk_idx(rank_row, k, kpad):
    pio = jax.lax.broadcasted_iota(jnp.int32, (kpad, T), 0).astype(jnp.float32)
    jio = jax.lax.broadcasted_iota(jnp.int32, (kpad, T), 1)
    onehot = jnp.where(rank_row == pio, jio, 0)
    return jnp.sum(onehot, axis=1)[:k].astype(jnp.int32)


def _sortable(u):
    return jnp.where(u >= 0, u, jnp.bitwise_xor(jnp.bitwise_not(u), _imin()))


def _unsortable(ks):
    u = jnp.where(ks >= 0, ks, jnp.bitwise_not(jnp.bitwise_xor(ks, _imin())))
    return jax.lax.bitcast_convert_type(u, jnp.float32)


NPHASE = 15


def _rank_kernel_body(cas_ref, act_ref, vs_ref, ie_ref, ib_ref, iha_ref, ihb_ref,
                      rd_s, ra_s, ki_s, ko_s, ri_s, ro_s):
    ph = pl.program_id(1)

    @pl.when(ph == 0)
    def _():
        act = _tree_reduce_c(cas_ref[...])
        a = act + 0.1 * jnp.roll(act, -1, axis=1) + 0.1 * jnp.roll(act, 1, axis=1) \
            + 0.02 * jnp.roll(act, -2, axis=1) + 0.02 * jnp.roll(act, 2, axis=1)
        act_ref[0] = a

    for c in range(NCH):
        @pl.when(ph == 1 + c)
        def _(c=c):
            rd, ra = _rank_chunk(act_ref[0], c * CH, True)
            rd_s[0, c * CH:(c + 1) * CH] = rd[0]
            ra_s[0, c * CH:(c + 1) * CH] = ra[0]

    @pl.when(ph == 5)
    def _():
        a = act_ref[0]
        ra = ra_s[...]
        va = jnp.sum(jnp.where(ra == float(T // 2 - 1), a, 0.0))
        vb = jnp.sum(jnp.where(ra == float(T // 2), a, 0.0))
        med = (va + vb) * 0.5
        ie_ref[0, 0, :] = _topk_idx(rd_s[...], K_EASY, 512)
        ib_ref[0, 0, :] = _topk_idx(ra, K_EASY, 512)
        binm = jnp.where(a > med, 1.0, 0.0)
        ki_s[...] = a * (_erode(binm, 3) - _erode(binm, 7))
        ko_s[...] = a * (_dilate(binm, 7) - _dilate(binm, 3))

    for c in range(NCH):
        @pl.when(ph == 6 + c)
        def _(c=c):
            rd, _unused = _rank_chunk(ki_s[...], c * CH, False)
            ri_s[0, c * CH:(c + 1) * CH] = rd[0]

    for c in range(NCH):
        @pl.when(ph == 10 + c)
        def _(c=c):
            rd, _unused = _rank_chunk(ko_s[...], c * CH, False)
            ro_s[0, c * CH:(c + 1) * CH] = rd[0]

    @pl.when(ph == 14)
    def _():
        iha_ref[0, 0, :] = _topk_idx(ri_s[...], K_HARD, 128)
        ihb_ref[0, 0, :] = _topk_idx(ro_s[...], K_HARD, 128)
        cas = cas_ref[0]
        u = jax.lax.bitcast_convert_type(cas, jnp.int32)
        q = jnp.bitwise_xor(_sortable(u), _imin())
        high = jnp.zeros((1, C), jnp.int32)
        for bit in range(31, -1, -1):
            cand = jnp.bitwise_or(high, jnp.int32(1) << bit)
            mask = jnp.int32(-1) << bit
            qm = jnp.bitwise_and(q, mask)
            cnt = jnp.sum(jnp.where(
                jnp.bitwise_xor(qm, _imin()) >= jnp.bitwise_xor(cand, _imin()), 1.0, 0.0),
                axis=0, keepdims=True)
            high = jnp.where(cnt >= float(K_EASY), cand, high)
        vstar = _unsortable(jnp.bitwise_xor(high, _imin()))
        gtm = cas > vstar
        cnt_gt = jnp.sum(jnp.where(gtm, 1.0, 0.0), axis=0, keepdims=True)
        sum_gt = jnp.sum(jnp.where(gtm, cas, 0.0), axis=0, keepdims=True)
        top_mean = (sum_gt + (float(K_EASY) - cnt_gt) * vstar) / float(K_EASY)
        m = jnp.max(top_mean, axis=1, keepdims=True)
        e = jnp.exp(top_mean - m)
        vs_ref[0] = e / jnp.sum(e, axis=1, keepdims=True)


def _rank_kernel(cas):
    f32 = jnp.float32
    return pl.pallas_call(
        _rank_kernel_body,
        grid=(B, NPHASE),
        in_specs=[pl.BlockSpec((1, T, C), lambda b, p: (b, 0, 0))],
        out_specs=[
            pl.BlockSpec((1, 1, T), lambda b, p: (b, 0, 0)),
            pl.BlockSpec((1, 1, C), lambda b, p: (b, 0, 0)),
            pl.BlockSpec((1, 1, K_EASY), lambda b, p: (b, 0, 0)),
            pl.BlockSpec((1, 1, K_EASY), lambda b, p: (b, 0, 0)),
            pl.BlockSpec((1, 1, K_HARD), lambda b, p: (b, 0, 0)),
            pl.BlockSpec((1, 1, K_HARD), lambda b, p: (b, 0, 0)),
        ],
        out_shape=[
            jax.ShapeDtypeStruct((B, 1, T), f32),
            jax.ShapeDtypeStruct((B, 1, C), f32),
            jax.ShapeDtypeStruct((B, 1, K_EASY), jnp.int32),
            jax.ShapeDtypeStruct((B, 1, K_EASY), jnp.int32),
            jax.ShapeDtypeStruct((B, 1, K_HARD), jnp.int32),
            jax.ShapeDtypeStruct((B, 1, K_HARD), jnp.int32),
        ],
        scratch_shapes=[pltpu.VMEM((1, T), f32) for _ in range(6)],
    )(cas)


def _gather_body(idx_ref, emb_ref, oe_ref, ob_ref, oha_ref, ohb_ref):
    b = pl.program_id(0)

    def copy_rows(o_ref, base, count):
        def body(j, carry):
            ix = idx_ref[b, base + j]
            o_ref[0, pl.ds(j, 1), :] = emb_ref[0, pl.ds(ix, 1), :]
            return carry
        jax.lax.fori_loop(0, count, body, 0)

    copy_rows(oe_ref, 0, K_EASY)
    copy_rows(ob_ref, K_EASY, K_EASY)
    copy_rows(oha_ref, 2 * K_EASY, K_HARD)
    copy_rows(ohb_ref, 2 * K_EASY + K_HARD, K_HARD)


def _gather_kernel(emb_bto, idx_all):
    grid_spec = pltpu.PrefetchScalarGridSpec(
        num_scalar_prefetch=1,
        grid=(B,),
        in_specs=[pl.BlockSpec((1, T, F), lambda b, idx: (b, 0, 0))],
        out_specs=[
            pl.BlockSpec((1, K_EASY, F), lambda b, idx: (b, 0, 0)),
            pl.BlockSpec((1, K_EASY, F), lambda b, idx: (b, 0, 0)),
            pl.BlockSpec((1, K_HARD, F), lambda b, idx: (b, 0, 0)),
            pl.BlockSpec((1, K_HARD, F), lambda b, idx: (b, 0, 0)),
        ],
    )
    return pl.pallas_call(
        _gather_body,
        grid_spec=grid_spec,
        out_shape=[
            jax.ShapeDtypeStruct((B, K_EASY, F), jnp.float32),
            jax.ShapeDtypeStruct((B, K_EASY, F), jnp.float32),
            jax.ShapeDtypeStruct((B, K_HARD, F), jnp.float32),
            jax.ShapeDtypeStruct((B, K_HARD, F), jnp.float32),
        ],
    )(idx_all, emb_bto)


def kernel(x, W_embed, b_embed, W_cls, b_cls):
    out_t = jnp.transpose(x, (0, 2, 1))
    emb = jax.lax.conv_general_dilated(out_t, W_embed, window_strides=(1,), padding=((1, 1),),
                                       dimension_numbers=("NCH", "OIH", "NCH"))
    embr = jax.nn.relu(emb + b_embed[None, :, None])
    cas = jax.lax.conv_general_dilated(embr, W_cls, window_strides=(1,), padding=((0, 0),),
                                       dimension_numbers=("NCH", "OIH", "NCH"))
    cas = jnp.transpose(cas, (0, 2, 1)) + b_cls[None, None, :]
    emb_bto = jnp.transpose(embr, (0, 2, 1))

    act_sm, vscores, ie, ib, iha, ihb = _rank_kernel(cas)
    act_sm = act_sm.reshape(B, T)
    vscores = vscores.reshape(B, C)
    idx_all = jnp.concatenate([ie, ib, iha, ihb], axis=2).reshape(B, 2 * K_EASY + 2 * K_HARD)
    easy_act, easy_bkg, hard_act, hard_bkg = _gather_kernel(emb_bto, idx_all)
    return (vscores, easy_act, easy_bkg, hard_act, hard_bkg, act_sm, cas)


# speedup vs baseline: 1.5841x; 1.0313x over previous
import jax
import jax.numpy as jnp
from jax.experimental import pallas as pl
from jax.experimental.pallas import tpu as pltpu

B = 8
T = 2048
F = 2048
C = 20
K_EASY = T // 5
K_HARD = T // 20
CH = 512
NCH = T // CH
IMIN = -2**31


def _imin():
    return jnp.full((), IMIN, jnp.int32)


def _tree_reduce_c(cas):
    lvl = [cas[:, :, i] for i in range(C)]
    while len(lvl) > 1:
        if len(lvl) % 2:
            lvl.append(jnp.zeros_like(lvl[0]))
        lvl = [lvl[2 * i] + lvl[2 * i + 1] for i in range(len(lvl) // 2)]
    return lvl[0]


def _shift0(v, s):
    if s > 0:
        return jnp.concatenate([jnp.zeros((v.shape[0], s), v.dtype), v[:, :-s]], axis=1)
    if s < 0:
        return jnp.concatenate([v[:, -s:], jnp.zeros((v.shape[0], -s), v.dtype)], axis=1)
    return v


def _erode(b, L):
    p = L // 2
    m = b
    for d in range(-p, p + 1):
        if d != 0:
            m = jnp.minimum(m, _shift0(b, d))
    return m


def _dilate(b, L):
    p = L // 2
    m = b
    for d in range(-p, p + 1):
        if d != 0:
            m = jnp.maximum(m, _shift0(b, d))
    return m


def _rank_chunk(key_row, i0, need_asc):
    ai = jnp.transpose(key_row[:, i0:i0 + CH])
    jio = jax.lax.broadcasted_iota(jnp.int32, (CH, T), 1)
    iio = jax.lax.broadcasted_iota(jnp.int32, (CH, T), 0) + i0
    eqb = (key_row == ai) & (jio < iio)
    rd = jnp.sum(jnp.where((key_row > ai) | eqb, 1.0, 0.0), axis=1).reshape(1, CH)
    if not need_asc:
        return rd, None
    ra = jnp.sum(jnp.where((key_row < ai) | eqb, 1.0, 0.0), axis=1).reshape(1, CH)
    return rd, ra


def _topk_idx(rank_row, k, kpad):
    pio = jax.lax.broadcasted_iota(jnp.int32, (kpad, T), 0).astype(jnp.float32)
    jio = jax.lax.broadcasted_iota(jnp.int32, (kpad, T), 1)
    onehot = jnp.where(rank_row == pio, jio, 0)
    return jnp.sum(onehot, axis=1)[:k].astype(jnp.int32)


def _sortable(u):
    return jnp.where(u >= 0, u, jnp.bitwise_xor(jnp.bitwise_not(u), _imin()))


def _unsortable(ks):
    u = jnp.where(ks >= 0, ks, jnp.bitwise_not(jnp.bitwise_xor(ks, _imin())))
    return jax.lax.bitcast_convert_type(u, jnp.float32)


NPHASE = 15


def _rank_kernel_body(cas_ref, act_ref, vs_ref, ie_ref, ib_ref, iha_ref, ihb_ref,
                      rd_s, ra_s, ki_s, ko_s, ri_s, ro_s):
    ph = pl.program_id(1)

    @pl.when(ph == 0)
    def _():
        act = _tree_reduce_c(cas_ref[...])
        a = act + 0.1 * jnp.roll(act, -1, axis=1) + 0.1 * jnp.roll(act, 1, axis=1) \
            + 0.02 * jnp.roll(act, -2, axis=1) + 0.02 * jnp.roll(act, 2, axis=1)
        act_ref[0] = a

    for c in range(NCH):
        @pl.when(ph == 1 + c)
        def _(c=c):
            rd, ra = _rank_chunk(act_ref[0], c * CH, True)
            rd_s[0, c * CH:(c + 1) * CH] = rd[0]
            ra_s[0, c * CH:(c + 1) * CH] = ra[0]

    @pl.when(ph == 5)
    def _():
        a = act_ref[0]
        ra = ra_s[...]
        va = jnp.sum(jnp.where(ra == float(T // 2 - 1), a, 0.0))
        vb = jnp.sum(jnp.where(ra == float(T // 2), a, 0.0))
        med = (va + vb) * 0.5
        ie_ref[0, 0, :] = _topk_idx(rd_s[...], K_EASY, 512)
        ib_ref[0, 0, :] = _topk_idx(ra, K_EASY, 512)
        binm = jnp.where(a > med, 1.0, 0.0)
        ki_s[...] = a * (_erode(binm, 3) - _erode(binm, 7))
        ko_s[...] = a * (_dilate(binm, 7) - _dilate(binm, 3))

    for c in range(NCH):
        @pl.when(ph == 6 + c)
        def _(c=c):
            rd, _unused = _rank_chunk(ki_s[...], c * CH, False)
            ri_s[0, c * CH:(c + 1) * CH] = rd[0]

    for c in range(NCH):
        @pl.when(ph == 10 + c)
        def _(c=c):
            rd, _unused = _rank_chunk(ko_s[...], c * CH, False)
            ro_s[0, c * CH:(c + 1) * CH] = rd[0]

    @pl.when(ph == 14)
    def _():
        iha_ref[0, 0, :] = _topk_idx(ri_s[...], K_HARD, 128)
        ihb_ref[0, 0, :] = _topk_idx(ro_s[...], K_HARD, 128)
        cas = cas_ref[0]
        u = jax.lax.bitcast_convert_type(cas, jnp.int32)
        q = jnp.bitwise_xor(_sortable(u), _imin())
        high = jnp.zeros((1, C), jnp.int32)
        for bit in range(31, -1, -1):
            cand = jnp.bitwise_or(high, jnp.int32(1) << bit)
            mask = jnp.int32(-1) << bit
            qm = jnp.bitwise_and(q, mask)
            cnt = jnp.sum(jnp.where(
                jnp.bitwise_xor(qm, _imin()) >= jnp.bitwise_xor(cand, _imin()), 1.0, 0.0),
                axis=0, keepdims=True)
            high = jnp.where(cnt >= float(K_EASY), cand, high)
        vstar = _unsortable(jnp.bitwise_xor(high, _imin()))
        gtm = cas > vstar
        cnt_gt = jnp.sum(jnp.where(gtm, 1.0, 0.0), axis=0, keepdims=True)
        sum_gt = jnp.sum(jnp.where(gtm, cas, 0.0), axis=0, keepdims=True)
        top_mean = (sum_gt + (float(K_EASY) - cnt_gt) * vstar) / float(K_EASY)
        m = jnp.max(top_mean, axis=1, keepdims=True)
        e = jnp.exp(top_mean - m)
        vs_ref[0] = e / jnp.sum(e, axis=1, keepdims=True)


def _rank_kernel(cas):
    f32 = jnp.float32
    return pl.pallas_call(
        _rank_kernel_body,
        grid=(B, NPHASE),
        in_specs=[pl.BlockSpec((1, T, C), lambda b, p: (b, 0, 0))],
        out_specs=[
            pl.BlockSpec((1, 1, T), lambda b, p: (b, 0, 0)),
            pl.BlockSpec((1, 1, C), lambda b, p: (b, 0, 0)),
            pl.BlockSpec((1, 1, K_EASY), lambda b, p: (b, 0, 0)),
            pl.BlockSpec((1, 1, K_EASY), lambda b, p: (b, 0, 0)),
            pl.BlockSpec((1, 1, K_HARD), lambda b, p: (b, 0, 0)),
            pl.BlockSpec((1, 1, K_HARD), lambda b, p: (b, 0, 0)),
        ],
        out_shape=[
            jax.ShapeDtypeStruct((B, 1, T), f32),
            jax.ShapeDtypeStruct((B, 1, C), f32),
            jax.ShapeDtypeStruct((B, 1, K_EASY), jnp.int32),
            jax.ShapeDtypeStruct((B, 1, K_EASY), jnp.int32),
            jax.ShapeDtypeStruct((B, 1, K_HARD), jnp.int32),
            jax.ShapeDtypeStruct((B, 1, K_HARD), jnp.int32),
        ],
        scratch_shapes=[pltpu.VMEM((1, T), f32) for _ in range(6)],
    )(cas)


def _gather_body(idx_ref, emb_ref, oe_ref, ob_ref, oha_ref, ohb_ref):
    b = pl.program_id(0)

    def copy_rows(o_ref, base, count):
        UNROLL = 2
        def body(j, carry):
            for u in range(UNROLL):
                ix = idx_ref[b, base + UNROLL * j + u]
                o_ref[0, pl.ds(UNROLL * j + u, 1), :] = emb_ref[0, pl.ds(ix, 1), :]
            return carry
        jax.lax.fori_loop(0, count // UNROLL, body, 0)
        for r in range(count - count % UNROLL, count):
            ix = idx_ref[b, base + r]
            o_ref[0, pl.ds(r, 1), :] = emb_ref[0, pl.ds(ix, 1), :]

    copy_rows(oe_ref, 0, K_EASY)
    copy_rows(ob_ref, K_EASY, K_EASY)
    copy_rows(oha_ref, 2 * K_EASY, K_HARD)
    copy_rows(ohb_ref, 2 * K_EASY + K_HARD, K_HARD)


def _gather_kernel(emb_bto, idx_all):
    grid_spec = pltpu.PrefetchScalarGridSpec(
        num_scalar_prefetch=1,
        grid=(B,),
        in_specs=[pl.BlockSpec((1, T, F), lambda b, idx: (b, 0, 0))],
        out_specs=[
            pl.BlockSpec((1, K_EASY, F), lambda b, idx: (b, 0, 0)),
            pl.BlockSpec((1, K_EASY, F), lambda b, idx: (b, 0, 0)),
            pl.BlockSpec((1, K_HARD, F), lambda b, idx: (b, 0, 0)),
            pl.BlockSpec((1, K_HARD, F), lambda b, idx: (b, 0, 0)),
        ],
    )
    return pl.pallas_call(
        _gather_body,
        grid_spec=grid_spec,
        out_shape=[
            jax.ShapeDtypeStruct((B, K_EASY, F), jnp.float32),
            jax.ShapeDtypeStruct((B, K_EASY, F), jnp.float32),
            jax.ShapeDtypeStruct((B, K_HARD, F), jnp.float32),
            jax.ShapeDtypeStruct((B, K_HARD, F), jnp.float32),
        ],
    )(idx_all, emb_bto)


def kernel(x, W_embed, b_embed, W_cls, b_cls):
    out_t = jnp.transpose(x, (0, 2, 1))
    emb = jax.lax.conv_general_dilated(out_t, W_embed, window_strides=(1,), padding=((1, 1),),
                                       dimension_numbers=("NCH", "OIH", "NCH"))
    embr = jax.nn.relu(emb + b_embed[None, :, None])
    cas = jax.lax.conv_general_dilated(embr, W_cls, window_strides=(1,), padding=((0, 0),),
                                       dimension_numbers=("NCH", "OIH", "NCH"))
    cas = jnp.transpose(cas, (0, 2, 1)) + b_cls[None, None, :]
    emb_bto = jnp.transpose(embr, (0, 2, 1))

    act_sm, vscores, ie, ib, iha, ihb = _rank_kernel(cas)
    act_sm = act_sm.reshape(B, T)
    vscores = vscores.reshape(B, C)
    idx_all = jnp.concatenate([ie, ib, iha, ihb], axis=2).reshape(B, 2 * K_EASY + 2 * K_HARD)
    easy_act, easy_bkg, hard_act, hard_bkg = _gather_kernel(emb_bto, idx_all)
    return (vscores, easy_act, easy_bkg, hard_act, hard_bkg, act_sm, cas)
